# baseline (speedup 1.0000x reference)
import functools

import jax
import jax.numpy as jnp
from jax import lax
from jax.experimental import pallas as pl
from jax.experimental.pallas import tpu as pltpu
from jax.experimental.pallas import tpu_sc as plsc

_SEQ_LEN = 200
_GROUPS_PER_BLOCK = 5
_ROWS_PER_WORKER = 8
_SC_WORKERS = 8
_PADDED_GROUPS = _SC_WORKERS * _ROWS_PER_WORKER


def _sc_gather_renorm(idx_hbm, table_hbm, out_hbm, idx_v, rows_v, sem):
    d = table_hbm.shape[1]
    n_chunks = d // 16
    num_cores = plsc.get_sparse_core_info().num_cores
    wid = lax.axis_index("s") * num_cores + lax.axis_index("c")

    @pl.when(wid < _SC_WORKERS)
    def _():
        base = wid * _ROWS_PER_WORKER
        pltpu.sync_copy(idx_hbm.at[pl.ds(base, _ROWS_PER_WORKER)], idx_v)
        pltpu.async_copy(table_hbm.at[idx_v], rows_v, sem).wait()
        for r in range(_ROWS_PER_WORKER):
            def _ss_body(c, acc, r=r):
                v = rows_v[r, pl.ds(c * 16, 16)]
                return acc + v * v
            acc = lax.fori_loop(0, n_chunks, _ss_body,
                                jnp.zeros((16,), jnp.float32))
            ss = acc[0]
            for lane in range(1, 16):
                ss = ss + acc[lane]
            y = lax.bitcast_convert_type(
                jnp.int32(0x5F3759DF)
                - (lax.bitcast_convert_type(ss, jnp.int32) >> 1),
                jnp.float32)
            for _ in range(4):
                y = y * (1.5 - 0.5 * ss * y * y)
            scale = jnp.full((16,), jnp.where(ss > 1.0, y, 1.0),
                             jnp.float32)

            def _scale_body(c, _, r=r, scale=scale):
                sl = pl.ds(c * 16, 16)
                rows_v[r, sl] = rows_v[r, sl] * scale
                return 0
            lax.fori_loop(0, n_chunks, _scale_body, 0)
        pltpu.sync_copy(rows_v, out_hbm.at[pl.ds(base, _ROWS_PER_WORKER)])


def _add_emb_kernel(x_ref, e_ref, o_ref, *, groups):
    p = pl.program_id(0)
    for h in range(groups):
        row = e_ref[pl.ds(p * groups + h, 1), :]
        sl = pl.ds(h * _SEQ_LEN, _SEQ_LEN)
        o_ref[sl] = x_ref[sl] + row


def kernel(x, num_people, table):
    total, b, d = x.shape
    blk = _SEQ_LEN * _GROUPS_PER_BLOCK
    n_blocks = total // blk
    n_groups = total // _SEQ_LEN
    idx = jnp.minimum(
        jnp.minimum(jnp.arange(_PADDED_GROUPS, dtype=jnp.int32),
                    n_groups - 1),
        jnp.asarray(num_people, jnp.int32) - 1)

    emb = pl.kernel(
        _sc_gather_renorm,
        out_type=jax.ShapeDtypeStruct((_PADDED_GROUPS, d), jnp.float32),
        mesh=plsc.VectorSubcoreMesh(core_axis_name="c", subcore_axis_name="s"),
        scratch_types=[
            pltpu.VMEM((_ROWS_PER_WORKER,), jnp.int32),
            pltpu.VMEM((_ROWS_PER_WORKER, d), jnp.float32),
            pltpu.SemaphoreType.DMA,
        ],
    )(idx, table)

    grid_spec = pl.GridSpec(
        grid=(n_blocks,),
        in_specs=[
            pl.BlockSpec((blk, b, d), lambda p: (p, 0, 0)),
            pl.BlockSpec((_PADDED_GROUPS, d), lambda p: (0, 0)),
        ],
        out_specs=pl.BlockSpec((blk, b, d), lambda p: (p, 0, 0)),
    )
    return pl.pallas_call(
        functools.partial(_add_emb_kernel, groups=_GROUPS_PER_BLOCK),
        grid_spec=grid_spec,
        out_shape=jax.ShapeDtypeStruct(x.shape, x.dtype),
    )(x, emb)


# speedup vs baseline: 1.0265x; 1.0265x over previous
import functools

import jax
import jax.numpy as jnp
from jax import lax
from jax.experimental import pallas as pl
from jax.experimental.pallas import tpu as pltpu
from jax.experimental.pallas import tpu_sc as plsc

_SEQ_LEN = 200
_GROUPS_PER_BLOCK = 5
_ROWS_PER_WORKER = 2
_SC_WORKERS = 32
_PADDED_GROUPS = _SC_WORKERS * _ROWS_PER_WORKER


def _sc_gather_renorm(idx_hbm, table_hbm, out_hbm, idx_v, rows_v, sem):
    d = table_hbm.shape[1]
    n_chunks = d // 16
    num_cores = plsc.get_sparse_core_info().num_cores
    wid = lax.axis_index("s") * num_cores + lax.axis_index("c")

    pltpu.sync_copy(idx_hbm.at[wid], idx_v)
    pltpu.async_copy(table_hbm.at[idx_v], rows_v, sem).wait()
    for r in range(_ROWS_PER_WORKER):
        def _ss_body(c, acc, r=r):
            v = rows_v[r, pl.ds(c * 16, 16)]
            return acc + v * v
        acc = lax.fori_loop(0, n_chunks, _ss_body,
                            jnp.zeros((16,), jnp.float32))
        ss = acc[0]
        for lane in range(1, 16):
            ss = ss + acc[lane]
        y = lax.bitcast_convert_type(
            jnp.int32(0x5F3759DF)
            - (lax.bitcast_convert_type(ss, jnp.int32) >> 1),
            jnp.float32)
        for _ in range(4):
            y = y * (1.5 - 0.5 * ss * y * y)
        scale = jnp.full((16,), jnp.where(ss > 1.0, y, 1.0),
                         jnp.float32)

        def _scale_body(c, _, r=r, scale=scale):
            sl = pl.ds(c * 16, 16)
            rows_v[r, sl] = rows_v[r, sl] * scale
            return 0
        lax.fori_loop(0, n_chunks, _scale_body, 0)
    pltpu.sync_copy(rows_v, out_hbm.at[wid])


def _add_emb_kernel(x_ref, e_ref, o_ref, *, groups):
    p = pl.program_id(0)
    for h in range(groups):
        row = e_ref[pl.ds(p * groups + h, 1), :]
        sl = pl.ds(h * _SEQ_LEN, _SEQ_LEN)
        o_ref[sl] = x_ref[sl] + row


def kernel(x, num_people, table):
    total, b, d = x.shape
    blk = _SEQ_LEN * _GROUPS_PER_BLOCK
    n_blocks = total // blk
    n_groups = total // _SEQ_LEN
    idx = jnp.minimum(
        jnp.minimum(jnp.arange(_PADDED_GROUPS, dtype=jnp.int32),
                    n_groups - 1),
        jnp.asarray(num_people, jnp.int32) - 1)
    idx = idx.reshape(_SC_WORKERS, _ROWS_PER_WORKER)

    emb = pl.kernel(
        _sc_gather_renorm,
        out_type=jax.ShapeDtypeStruct(
            (_SC_WORKERS, _ROWS_PER_WORKER, d), jnp.float32),
        mesh=plsc.VectorSubcoreMesh(core_axis_name="c", subcore_axis_name="s"),
        scratch_types=[
            pltpu.VMEM((_ROWS_PER_WORKER,), jnp.int32),
            pltpu.VMEM((_ROWS_PER_WORKER, d), jnp.float32),
            pltpu.SemaphoreType.DMA,
        ],
    )(idx, table)
    emb = emb.reshape(_PADDED_GROUPS, d)

    grid_spec = pl.GridSpec(
        grid=(n_blocks,),
        in_specs=[
            pl.BlockSpec((blk, b, d), lambda p: (p, 0, 0)),
            pl.BlockSpec((_PADDED_GROUPS, d), lambda p: (0, 0)),
        ],
        out_specs=pl.BlockSpec((blk, b, d), lambda p: (p, 0, 0)),
    )
    return pl.pallas_call(
        functools.partial(_add_emb_kernel, groups=_GROUPS_PER_BLOCK),
        grid_spec=grid_spec,
        out_shape=jax.ShapeDtypeStruct(x.shape, x.dtype),
    )(x, emb)


# speedup vs baseline: 1.0307x; 1.0041x over previous
import functools

import jax
import jax.numpy as jnp
from jax import lax
from jax.experimental import pallas as pl
from jax.experimental.pallas import tpu as pltpu
from jax.experimental.pallas import tpu_sc as plsc

_SEQ_LEN = 200
_GROUPS_PER_BLOCK = 5
_ROWS_PER_WORKER = 2
_SC_WORKERS = 32
_PADDED_GROUPS = _SC_WORKERS * _ROWS_PER_WORKER


def _sc_gather_renorm(idx_hbm, table_hbm, out_hbm, idx_v, rows_v, sem):
    d = table_hbm.shape[1]
    n_chunks = d // 16
    num_cores = plsc.get_sparse_core_info().num_cores
    wid = lax.axis_index("s") * num_cores + lax.axis_index("c")

    pltpu.sync_copy(idx_hbm.at[wid], idx_v)
    pltpu.async_copy(table_hbm.at[idx_v], rows_v, sem).wait()
    for r in range(_ROWS_PER_WORKER):
        acc = jnp.zeros((16,), jnp.float32)
        for c in range(n_chunks):
            v = rows_v[r, pl.ds(c * 16, 16)]
            acc = acc + v * v
        ss = acc[0]
        for lane in range(1, 16):
            ss = ss + acc[lane]
        y = lax.bitcast_convert_type(
            jnp.int32(0x5F3759DF)
            - (lax.bitcast_convert_type(ss, jnp.int32) >> 1),
            jnp.float32)
        for _ in range(4):
            y = y * (1.5 - 0.5 * ss * y * y)
        scale = jnp.full((16,), jnp.where(ss > 1.0, y, 1.0),
                         jnp.float32)

        for c in range(n_chunks):
            sl = pl.ds(c * 16, 16)
            rows_v[r, sl] = rows_v[r, sl] * scale
    pltpu.sync_copy(rows_v, out_hbm.at[wid])


def _add_emb_kernel(x_ref, e_ref, o_ref, *, groups):
    p = pl.program_id(0)
    for h in range(groups):
        row = e_ref[pl.ds(p * groups + h, 1), :]
        sl = pl.ds(h * _SEQ_LEN, _SEQ_LEN)
        o_ref[sl] = x_ref[sl] + row


def kernel(x, num_people, table):
    total, b, d = x.shape
    blk = _SEQ_LEN * _GROUPS_PER_BLOCK
    n_blocks = total // blk
    n_groups = total // _SEQ_LEN
    idx = jnp.minimum(
        jnp.minimum(jnp.arange(_PADDED_GROUPS, dtype=jnp.int32),
                    n_groups - 1),
        jnp.asarray(num_people, jnp.int32) - 1)
    idx = idx.reshape(_SC_WORKERS, _ROWS_PER_WORKER)

    emb = pl.kernel(
        _sc_gather_renorm,
        out_type=jax.ShapeDtypeStruct(
            (_SC_WORKERS, _ROWS_PER_WORKER, d), jnp.float32),
        mesh=plsc.VectorSubcoreMesh(core_axis_name="c", subcore_axis_name="s"),
        scratch_types=[
            pltpu.VMEM((_ROWS_PER_WORKER,), jnp.int32),
            pltpu.VMEM((_ROWS_PER_WORKER, d), jnp.float32),
            pltpu.SemaphoreType.DMA,
        ],
    )(idx, table)
    emb = emb.reshape(_PADDED_GROUPS, d)

    grid_spec = pl.GridSpec(
        grid=(n_blocks,),
        in_specs=[
            pl.BlockSpec((blk, b, d), lambda p: (p, 0, 0)),
            pl.BlockSpec((_PADDED_GROUPS, d), lambda p: (0, 0)),
        ],
        out_specs=pl.BlockSpec((blk, b, d), lambda p: (p, 0, 0)),
    )
    return pl.pallas_call(
        functools.partial(_add_emb_kernel, groups=_GROUPS_PER_BLOCK),
        grid_spec=grid_spec,
        out_shape=jax.ShapeDtypeStruct(x.shape, x.dtype),
    )(x, emb)
